# baseline (speedup 1.0000x reference)
import functools

import jax
import jax.numpy as jnp
from jax import lax
from jax.experimental import pallas as pl
from jax.experimental.pallas import tpu as pltpu
from jax.experimental.pallas import tpu_sc as plsc

N_NODES = 10000
N_EDGES = 320000
N_GRAPHS = 256
ATOM_EMBED = 64
HIDDEN = 128
N_RBF = 16
MAX_RADIUS = 5.0

E_PAD = 327680
NC = 2
NS = 16
NW = NC * NS
EPW = N_EDGES // NW
EPC = N_EDGES // NC
RPT = N_NODES // NS


def _sc_gather_body(row_h, col_h, z_h, px_h, py_h, pz_h, d2_h, zc_h,
                    row_v, col_v, z_v, px_v, py_v, pz_v, d2_v, zc_v):
    c = lax.axis_index("c")
    s = lax.axis_index("s")
    wid = s * NC + c
    base = wid * EPW
    pltpu.sync_copy(row_h.at[pl.ds(base, EPW)], row_v)
    pltpu.sync_copy(col_h.at[pl.ds(base, EPW)], col_v)
    pltpu.sync_copy(z_h, z_v)
    pltpu.sync_copy(px_h, px_v)
    pltpu.sync_copy(py_h, py_v)
    pltpu.sync_copy(pz_h, pz_v)

    def body(i, carry):
        sl = pl.ds(i * 16, 16)
        r = row_v[sl]
        cc = col_v[sl]
        ax = plsc.load_gather(px_v, [r])
        bx = plsc.load_gather(px_v, [cc])
        ay = plsc.load_gather(py_v, [r])
        by = plsc.load_gather(py_v, [cc])
        az = plsc.load_gather(pz_v, [r])
        bz = plsc.load_gather(pz_v, [cc])
        dx = ax - bx
        dy = ay - by
        dz = az - bz
        d2_v[sl] = dx * dx + dy * dy + dz * dz
        zc_v[sl] = plsc.load_gather(z_v, [cc])
        return carry

    lax.fori_loop(0, EPW // 16, body, 0)
    pltpu.sync_copy(d2_v, d2_h.at[pl.ds(base, EPW)])
    pltpu.sync_copy(zc_v, zc_h.at[pl.ds(base, EPW)])


@functools.cache
def _make_sc_gather():
    mesh = plsc.VectorSubcoreMesh(core_axis_name="c", subcore_axis_name="s")
    return functools.partial(
        pl.kernel,
        mesh=mesh,
        out_type=(jax.ShapeDtypeStruct((E_PAD,), jnp.float32),
                  jax.ShapeDtypeStruct((E_PAD,), jnp.int32)),
        scratch_types=[
            pltpu.VMEM((EPW,), jnp.int32),
            pltpu.VMEM((EPW,), jnp.int32),
            pltpu.VMEM((N_NODES,), jnp.int32),
            pltpu.VMEM((N_NODES,), jnp.float32),
            pltpu.VMEM((N_NODES,), jnp.float32),
            pltpu.VMEM((N_NODES,), jnp.float32),
            pltpu.VMEM((EPW,), jnp.float32),
            pltpu.VMEM((EPW,), jnp.int32),
        ],
        compiler_params=pltpu.CompilerParams(needs_layout_passes=False),
    )(_sc_gather_body)


def _sc_gather(row, col, z, px, py, pz):
    return _make_sc_gather()(row, col, z, px, py, pz)


_CHUNK = 128
_NFULL = EPW // _CHUNK
_TAIL = EPW - _NFULL * _CHUNK


def _sc_scatter_body(row_h, msg_h, agg_h, idx0, msg0, idx1, msg1,
                     idxt_v, msgt_v, si0, sm0, si1, sm1, acc_sh):
    c = lax.axis_index("c")
    s = lax.axis_index("s")
    base = (c * NS + s) * EPW
    bufs = ((idx0, msg0, si0, sm0), (idx1, msg1, si1, sm1))

    def zbody(r, carry):
        for q in range(HIDDEN // 16):
            msg0[r, pl.ds(q * 16, 16)] = jnp.zeros((16,), jnp.float32)
        return carry

    lax.fori_loop(0, _CHUNK, zbody, 0)
    off = 0
    for nrows in (128, 128, 128, 128, 112):
        pltpu.sync_copy(msg0.at[pl.ds(0, nrows)],
                        acc_sh.at[pl.ds(s * 624 + off, nrows)])
        off += nrows

    @pl.when(s == NS - 1)
    def _():
        pltpu.sync_copy(msg0.at[pl.ds(0, 16)],
                        acc_sh.at[pl.ds(N_NODES - 16, 16)])

    plsc.subcore_barrier()

    def _start(j, b):
        idx_b, msg_b, si, sm = b
        o = base + j * _CHUNK
        pltpu.async_copy(row_h.at[pl.ds(o, _CHUNK)], idx_b, si)
        pltpu.async_copy(msg_h.at[pl.ds(o, _CHUNK), :], msg_b, sm)

    def _wait(b):
        idx_b, msg_b, si, sm = b
        pltpu.make_async_copy(row_h.at[pl.ds(base, _CHUNK)], idx_b, si).wait()
        pltpu.make_async_copy(msg_h.at[pl.ds(base, _CHUNK), :], msg_b, sm).wait()

    _start(0, bufs[0])
    _start(1, bufs[1])

    def body(jj, carry):
        for k in range(2):
            j = jj * 2 + k
            b = bufs[k]
            _wait(b)
            pltpu.sync_copy(b[1], acc_sh.at[b[0]], add=True)

            @pl.when(j + 2 < _NFULL)
            def _():
                _start(j + 2, b)
        return carry

    lax.fori_loop(0, _NFULL // 2, body, 0)
    o = base + _NFULL * _CHUNK
    pltpu.sync_copy(row_h.at[pl.ds(o, _TAIL)], idxt_v)
    pltpu.sync_copy(msg_h.at[pl.ds(o, _TAIL), :], msgt_v)
    pltpu.sync_copy(msgt_v, acc_sh.at[idxt_v], add=True)
    plsc.subcore_barrier()

    off = 0
    for nrows in (128, 128, 128, 128, 112):
        r0 = s * 624 + off
        pltpu.sync_copy(acc_sh.at[pl.ds(r0, nrows)], msg0.at[pl.ds(0, nrows)])
        pltpu.sync_copy(msg0.at[pl.ds(0, nrows)], agg_h.at[c, pl.ds(r0, nrows), :])
        off += nrows

    @pl.when(s == NS - 1)
    def _():
        r0 = N_NODES - 16
        pltpu.sync_copy(acc_sh.at[pl.ds(r0, 16)], msgt_v)
        pltpu.sync_copy(msgt_v, agg_h.at[c, pl.ds(r0, 16), :])


@functools.cache
def _make_sc_scatter():
    mesh = plsc.VectorSubcoreMesh(core_axis_name="c", subcore_axis_name="s")
    return functools.partial(
        pl.kernel,
        mesh=mesh,
        out_type=jax.ShapeDtypeStruct((NC, N_NODES, HIDDEN), jnp.float32),
        scratch_types=[
            pltpu.VMEM((_CHUNK,), jnp.int32),
            pltpu.VMEM((_CHUNK, HIDDEN), jnp.float32),
            pltpu.VMEM((_CHUNK,), jnp.int32),
            pltpu.VMEM((_CHUNK, HIDDEN), jnp.float32),
            pltpu.VMEM((_TAIL,), jnp.int32),
            pltpu.VMEM((_TAIL, HIDDEN), jnp.float32),
            pltpu.SemaphoreType.DMA,
            pltpu.SemaphoreType.DMA,
            pltpu.SemaphoreType.DMA,
            pltpu.SemaphoreType.DMA,
            pltpu.VMEM_SHARED((N_NODES, HIDDEN), jnp.float32),
        ],
        compiler_params=pltpu.CompilerParams(needs_layout_passes=False),
    )(_sc_scatter_body)


def _sc_scatter(row, msgs):
    return _make_sc_scatter()(row, msgs)


BE = 4096


def _edge_body(d2_ref, zc_ref, at_ref, mw1_ref, b1_ref, mw2_ref, b2_ref,
               gam_ref, cen_ref, msg_ref):
    d2 = d2_ref[...][None, :]
    elen = jnp.sqrt(d2 + 1e-12)
    diff = elen - cen_ref[...]
    rbfT = jnp.exp(-gam_ref[0, 0] * (diff * diff))
    zc = zc_ref[...][None, :]
    ohT = (zc == lax.broadcasted_iota(jnp.int32, (HIDDEN, BE), 0)
           ).astype(jnp.float32)
    xT = jnp.dot(at_ref[...].T, ohT, preferred_element_type=jnp.float32)
    msg_inT = jnp.concatenate([xT, rbfT], axis=0)
    pre = jnp.dot(mw1_ref[...].T, msg_inT,
                  preferred_element_type=jnp.float32)
    hT = jnp.maximum(pre + b1_ref[...], 0.0)
    out = lax.dot_general(hT, mw2_ref[...], (((0,), (0,)), ((), ())),
                          preferred_element_type=jnp.float32)
    msg_ref[...] = jnp.maximum(out + b2_ref[...], 0.0)


BN = 2000
NB = N_NODES // BN


def _node_body(z_ref, agg_ref, bat_ref, at_ref, nw1_ref, b1_ref, nw2_ref,
               b2_ref, rw1_ref, rb1_ref, rw2_ref, rb2_ref, out_ref, mol_ref):
    i = pl.program_id(0)
    zr = z_ref[0]
    ohzT = (zr == lax.broadcasted_iota(jnp.int32, (HIDDEN, BN), 0)
            ).astype(jnp.float32)
    xT = jnp.dot(at_ref[...].T, ohzT, preferred_element_type=jnp.float32)
    aggT = (agg_ref[0] + agg_ref[1]).T
    nfT = jnp.concatenate([xT, aggT], axis=0)
    pre = jnp.dot(nw1_ref[...].T, nfT, preferred_element_type=jnp.float32)
    h2T = jnp.maximum(pre + b1_ref[...], 0.0)
    nout = lax.dot_general(h2T, nw2_ref[...], (((0,), (0,)), ((), ())),
                           preferred_element_type=jnp.float32)
    nout = jnp.maximum(nout + b2_ref[...], 0.0)
    bt = bat_ref[0]
    ohb = (lax.broadcasted_iota(jnp.int32, (N_GRAPHS, BN), 0) == bt
           ).astype(jnp.float32)
    part = jnp.dot(ohb, nout, preferred_element_type=jnp.float32, precision=lax.Precision.HIGHEST)

    @pl.when(i == 0)
    def _():
        mol_ref[...] = part

    @pl.when(i > 0)
    def _():
        mol_ref[...] = mol_ref[...] + part

    @pl.when(i == NB - 1)
    def _():
        h3 = jnp.dot(mol_ref[...], rw1_ref[...],
                     preferred_element_type=jnp.float32)
        h3 = jnp.maximum(h3 + rb1_ref[...], 0.0)
        o = jnp.dot(h3, rw2_ref[...], preferred_element_type=jnp.float32)
        out_ref[...] = o + rb2_ref[...]


def kernel(pos, z, batch, edge_index, atom_table, gamma,
           msg_W1, msg_b1, msg_W2, msg_b2,
           node_W1, node_b1, node_W2, node_b2,
           ro_W1, ro_b1, ro_W2, ro_b2):
    row = edge_index[0]
    col = edge_index[1]
    px = pos[:, 0]
    py = pos[:, 1]
    pz = pos[:, 2]

    d2, zcol = _sc_gather(row, col, z, px, py, pz)

    at_pad = jnp.pad(atom_table, ((0, HIDDEN - atom_table.shape[0]), (0, 0)))
    centers = jnp.linspace(0.0, MAX_RADIUS, N_RBF,
                           dtype=jnp.float32).reshape(N_RBF, 1)
    msgs = pl.pallas_call(
        _edge_body,
        grid=(E_PAD // BE,),
        in_specs=[
            pl.BlockSpec((BE,), lambda i: (i,)),
            pl.BlockSpec((BE,), lambda i: (i,)),
            pl.BlockSpec((HIDDEN, ATOM_EMBED), lambda i: (0, 0)),
            pl.BlockSpec((ATOM_EMBED + N_RBF, HIDDEN), lambda i: (0, 0)),
            pl.BlockSpec((HIDDEN, 1), lambda i: (0, 0)),
            pl.BlockSpec((HIDDEN, HIDDEN), lambda i: (0, 0)),
            pl.BlockSpec((1, HIDDEN), lambda i: (0, 0)),
            pl.BlockSpec((1, 1), lambda i: (0, 0)),
            pl.BlockSpec((N_RBF, 1), lambda i: (0, 0)),
        ],
        out_specs=pl.BlockSpec((BE, HIDDEN), lambda i: (i, 0)),
        out_shape=jax.ShapeDtypeStruct((E_PAD, HIDDEN), jnp.float32),
        compiler_params=pltpu.CompilerParams(
            dimension_semantics=("arbitrary",)),
    )(d2, zcol, at_pad, msg_W1, msg_b1.reshape(HIDDEN, 1), msg_W2,
      msg_b2.reshape(1, HIDDEN), gamma.reshape(1, 1), centers)

    agg2 = _sc_scatter(row, msgs)

    out = pl.pallas_call(
        _node_body,
        grid=(NB,),
        in_specs=[
            pl.BlockSpec((1, 1, BN), lambda i: (i, 0, 0)),
            pl.BlockSpec((NC, BN, HIDDEN), lambda i: (0, i, 0)),
            pl.BlockSpec((1, 1, BN), lambda i: (i, 0, 0)),
            pl.BlockSpec((HIDDEN, ATOM_EMBED), lambda i: (0, 0)),
            pl.BlockSpec((ATOM_EMBED + HIDDEN, HIDDEN), lambda i: (0, 0)),
            pl.BlockSpec((HIDDEN, 1), lambda i: (0, 0)),
            pl.BlockSpec((HIDDEN, HIDDEN), lambda i: (0, 0)),
            pl.BlockSpec((1, HIDDEN), lambda i: (0, 0)),
            pl.BlockSpec((HIDDEN, HIDDEN // 2), lambda i: (0, 0)),
            pl.BlockSpec((1, HIDDEN // 2), lambda i: (0, 0)),
            pl.BlockSpec((HIDDEN // 2, 1), lambda i: (0, 0)),
            pl.BlockSpec((1, 1), lambda i: (0, 0)),
        ],
        out_specs=pl.BlockSpec((N_GRAPHS, 1), lambda i: (0, 0)),
        out_shape=jax.ShapeDtypeStruct((N_GRAPHS, 1), jnp.float32),
        scratch_shapes=[pltpu.VMEM((N_GRAPHS, HIDDEN), jnp.float32)],
        compiler_params=pltpu.CompilerParams(
            dimension_semantics=("arbitrary",)),
    )(z.reshape(NB, 1, BN), agg2, batch.reshape(NB, 1, BN), at_pad,
      node_W1, node_b1.reshape(HIDDEN, 1), node_W2,
      node_b2.reshape(1, HIDDEN), ro_W1, ro_b1.reshape(1, HIDDEN // 2),
      ro_W2, ro_b2.reshape(1, 1))

    return out.reshape(N_GRAPHS)


# speedup vs baseline: 1.0586x; 1.0586x over previous
import functools

import jax
import jax.numpy as jnp
from jax import lax
from jax.experimental import pallas as pl
from jax.experimental.pallas import tpu as pltpu
from jax.experimental.pallas import tpu_sc as plsc

N_NODES = 10000
N_EDGES = 320000
N_GRAPHS = 256
ATOM_EMBED = 64
HIDDEN = 128
N_RBF = 16
MAX_RADIUS = 5.0

E_PAD = 327680
NC = 2
NS = 16
NW = NC * NS
EPW = N_EDGES // NW
EPC = N_EDGES // NC
RPT = N_NODES // NS


def _sc_gather_body(row_h, col_h, z_h, px_h, py_h, pz_h, d2_h, zc_h,
                    row_v, col_v, z_v, px_v, py_v, pz_v, d2_v, zc_v):
    c = lax.axis_index("c")
    s = lax.axis_index("s")
    wid = s * NC + c
    base = wid * EPW
    pltpu.sync_copy(row_h.at[pl.ds(base, EPW)], row_v)
    pltpu.sync_copy(col_h.at[pl.ds(base, EPW)], col_v)
    pltpu.sync_copy(z_h, z_v)
    pltpu.sync_copy(px_h, px_v)
    pltpu.sync_copy(py_h, py_v)
    pltpu.sync_copy(pz_h, pz_v)

    def body(i, carry):
        sl = pl.ds(i * 16, 16)
        r = row_v[sl]
        cc = col_v[sl]
        ax = plsc.load_gather(px_v, [r])
        bx = plsc.load_gather(px_v, [cc])
        ay = plsc.load_gather(py_v, [r])
        by = plsc.load_gather(py_v, [cc])
        az = plsc.load_gather(pz_v, [r])
        bz = plsc.load_gather(pz_v, [cc])
        dx = ax - bx
        dy = ay - by
        dz = az - bz
        d2_v[sl] = dx * dx + dy * dy + dz * dz
        zc_v[sl] = plsc.load_gather(z_v, [cc])
        return carry

    lax.fori_loop(0, EPW // 16, body, 0)
    pltpu.sync_copy(d2_v, d2_h.at[pl.ds(base, EPW)])
    pltpu.sync_copy(zc_v, zc_h.at[pl.ds(base, EPW)])


@functools.cache
def _make_sc_gather():
    mesh = plsc.VectorSubcoreMesh(core_axis_name="c", subcore_axis_name="s")
    return functools.partial(
        pl.kernel,
        mesh=mesh,
        out_type=(jax.ShapeDtypeStruct((E_PAD,), jnp.float32),
                  jax.ShapeDtypeStruct((E_PAD,), jnp.int32)),
        scratch_types=[
            pltpu.VMEM((EPW,), jnp.int32),
            pltpu.VMEM((EPW,), jnp.int32),
            pltpu.VMEM((N_NODES,), jnp.int32),
            pltpu.VMEM((N_NODES,), jnp.float32),
            pltpu.VMEM((N_NODES,), jnp.float32),
            pltpu.VMEM((N_NODES,), jnp.float32),
            pltpu.VMEM((EPW,), jnp.float32),
            pltpu.VMEM((EPW,), jnp.int32),
        ],
        compiler_params=pltpu.CompilerParams(needs_layout_passes=False),
    )(_sc_gather_body)


def _sc_gather(row, col, z, px, py, pz):
    return _make_sc_gather()(row, col, z, px, py, pz)


E_HALF = N_EDGES // 2
EH_PAD = E_PAD // 2
SEPW = E_HALF // NW
_CHUNK = 128
_NFULL = SEPW // _CHUNK
_TAIL = SEPW - _NFULL * _CHUNK


def _sc_scatter_body(row_h, msg_h, agg_h, idx0, msg0, idx1, msg1,
                     idxt_v, msgt_v, si0, sm0, si1, sm1, acc_sh):
    c = lax.axis_index("c")
    s = lax.axis_index("s")
    base = (c * NS + s) * SEPW
    bufs = ((idx0, msg0, si0, sm0), (idx1, msg1, si1, sm1))

    def zbody(r, carry):
        for q in range(HIDDEN // 16):
            msg0[r, pl.ds(q * 16, 16)] = jnp.zeros((16,), jnp.float32)
        return carry

    lax.fori_loop(0, _CHUNK, zbody, 0)
    off = 0
    for nrows in (128, 128, 128, 128, 112):
        pltpu.sync_copy(msg0.at[pl.ds(0, nrows)],
                        acc_sh.at[pl.ds(s * 624 + off, nrows)])
        off += nrows

    @pl.when(s == NS - 1)
    def _():
        pltpu.sync_copy(msg0.at[pl.ds(0, 16)],
                        acc_sh.at[pl.ds(N_NODES - 16, 16)])

    plsc.subcore_barrier()

    def _start(j, b):
        idx_b, msg_b, si, sm = b
        o = base + j * _CHUNK
        pltpu.async_copy(row_h.at[pl.ds(o, _CHUNK)], idx_b, si)
        pltpu.async_copy(msg_h.at[pl.ds(o, _CHUNK), :], msg_b, sm)

    def _wait(b):
        idx_b, msg_b, si, sm = b
        pltpu.make_async_copy(row_h.at[pl.ds(base, _CHUNK)], idx_b, si).wait()
        pltpu.make_async_copy(msg_h.at[pl.ds(base, _CHUNK), :], msg_b, sm).wait()

    _start(0, bufs[0])
    _start(1, bufs[1])

    def body(jj, carry):
        for k in range(2):
            j = jj * 2 + k
            b = bufs[k]
            _wait(b)
            pltpu.sync_copy(b[1], acc_sh.at[b[0]], add=True)

            @pl.when(j + 2 < _NFULL)
            def _():
                _start(j + 2, b)
        return carry

    lax.fori_loop(0, _NFULL // 2, body, 0)
    if _NFULL % 2:
        b = bufs[0]
        _wait(b)
        pltpu.sync_copy(b[1], acc_sh.at[b[0]], add=True)
    o = base + _NFULL * _CHUNK
    pltpu.sync_copy(row_h.at[pl.ds(o, _TAIL)], idxt_v)
    pltpu.sync_copy(msg_h.at[pl.ds(o, _TAIL), :], msgt_v)
    pltpu.sync_copy(msgt_v, acc_sh.at[idxt_v], add=True)
    plsc.subcore_barrier()

    off = 0
    for nrows in (128, 128, 128, 128, 112):
        r0 = s * 624 + off
        pltpu.sync_copy(acc_sh.at[pl.ds(r0, nrows)], msg0.at[pl.ds(0, nrows)])
        pltpu.sync_copy(msg0.at[pl.ds(0, nrows)], agg_h.at[c, pl.ds(r0, nrows), :])
        off += nrows

    @pl.when(s == NS - 1)
    def _():
        r0 = N_NODES - 16
        pltpu.sync_copy(acc_sh.at[pl.ds(r0, 16)], msg0.at[pl.ds(0, 16)])
        pltpu.sync_copy(msg0.at[pl.ds(0, 16)], agg_h.at[c, pl.ds(r0, 16), :])


@functools.cache
def _make_sc_scatter():
    mesh = plsc.VectorSubcoreMesh(core_axis_name="c", subcore_axis_name="s")
    return functools.partial(
        pl.kernel,
        mesh=mesh,
        out_type=jax.ShapeDtypeStruct((NC, N_NODES, HIDDEN), jnp.float32),
        scratch_types=[
            pltpu.VMEM((_CHUNK,), jnp.int32),
            pltpu.VMEM((_CHUNK, HIDDEN), jnp.float32),
            pltpu.VMEM((_CHUNK,), jnp.int32),
            pltpu.VMEM((_CHUNK, HIDDEN), jnp.float32),
            pltpu.VMEM((_TAIL,), jnp.int32),
            pltpu.VMEM((_TAIL, HIDDEN), jnp.float32),
            pltpu.SemaphoreType.DMA,
            pltpu.SemaphoreType.DMA,
            pltpu.SemaphoreType.DMA,
            pltpu.SemaphoreType.DMA,
            pltpu.VMEM_SHARED((N_NODES, HIDDEN), jnp.float32),
        ],
        compiler_params=pltpu.CompilerParams(needs_layout_passes=False),
    )(_sc_scatter_body)


def _sc_scatter(row, msgs):
    return _make_sc_scatter()(row, msgs)


BE = 4096


def _edge_body(d2_ref, zc_ref, at_ref, mw1_ref, b1_ref, mw2_ref, b2_ref,
               gam_ref, cen_ref, msg_ref):
    d2 = d2_ref[...][None, :]
    elen = jnp.sqrt(d2 + 1e-12)
    diff = elen - cen_ref[...]
    rbfT = jnp.exp(-gam_ref[0, 0] * (diff * diff))
    zc = zc_ref[...][None, :]
    ohT = (zc == lax.broadcasted_iota(jnp.int32, (HIDDEN, BE), 0)
           ).astype(jnp.float32)
    xT = jnp.dot(at_ref[...].T, ohT, preferred_element_type=jnp.float32)
    msg_inT = jnp.concatenate([xT, rbfT], axis=0)
    pre = jnp.dot(mw1_ref[...].T, msg_inT,
                  preferred_element_type=jnp.float32)
    hT = jnp.maximum(pre + b1_ref[...], 0.0)
    out = lax.dot_general(hT, mw2_ref[...], (((0,), (0,)), ((), ())),
                          preferred_element_type=jnp.float32)
    msg_ref[...] = jnp.maximum(out + b2_ref[...], 0.0)


BN = 2000
NB = N_NODES // BN


def _node_body(z_ref, agg_ref, aggb_ref, bat_ref, at_ref, nw1_ref, b1_ref,
               nw2_ref, b2_ref, rw1_ref, rb1_ref, rw2_ref, rb2_ref,
               out_ref, mol_ref):
    i = pl.program_id(0)
    zr = z_ref[0]
    ohzT = (zr == lax.broadcasted_iota(jnp.int32, (HIDDEN, BN), 0)
            ).astype(jnp.float32)
    xT = jnp.dot(at_ref[...].T, ohzT, preferred_element_type=jnp.float32)
    aggT = ((agg_ref[0] + agg_ref[1])
            + (aggb_ref[0] + aggb_ref[1])).T
    nfT = jnp.concatenate([xT, aggT], axis=0)
    pre = jnp.dot(nw1_ref[...].T, nfT, preferred_element_type=jnp.float32)
    h2T = jnp.maximum(pre + b1_ref[...], 0.0)
    nout = lax.dot_general(h2T, nw2_ref[...], (((0,), (0,)), ((), ())),
                           preferred_element_type=jnp.float32)
    nout = jnp.maximum(nout + b2_ref[...], 0.0)
    bt = bat_ref[0]
    ohb = (lax.broadcasted_iota(jnp.int32, (N_GRAPHS, BN), 0) == bt
           ).astype(jnp.float32)
    part = jnp.dot(ohb, nout, preferred_element_type=jnp.float32, precision=lax.Precision.HIGHEST)

    @pl.when(i == 0)
    def _():
        mol_ref[...] = part

    @pl.when(i > 0)
    def _():
        mol_ref[...] = mol_ref[...] + part

    @pl.when(i == NB - 1)
    def _():
        h3 = jnp.dot(mol_ref[...], rw1_ref[...],
                     preferred_element_type=jnp.float32)
        h3 = jnp.maximum(h3 + rb1_ref[...], 0.0)
        o = jnp.dot(h3, rw2_ref[...], preferred_element_type=jnp.float32)
        out_ref[...] = o + rb2_ref[...]


def kernel(pos, z, batch, edge_index, atom_table, gamma,
           msg_W1, msg_b1, msg_W2, msg_b2,
           node_W1, node_b1, node_W2, node_b2,
           ro_W1, ro_b1, ro_W2, ro_b2):
    row = edge_index[0]
    col = edge_index[1]
    px = pos[:, 0]
    py = pos[:, 1]
    pz = pos[:, 2]

    d2, zcol = _sc_gather(row, col, z, px, py, pz)

    at_pad = jnp.pad(atom_table, ((0, HIDDEN - atom_table.shape[0]), (0, 0)))
    centers = jnp.linspace(0.0, MAX_RADIUS, N_RBF,
                           dtype=jnp.float32).reshape(N_RBF, 1)

    def edge_half(d2h, zch):
        return pl.pallas_call(
            _edge_body,
            grid=(EH_PAD // BE,),
            in_specs=[
                pl.BlockSpec((BE,), lambda i: (i,)),
                pl.BlockSpec((BE,), lambda i: (i,)),
                pl.BlockSpec((HIDDEN, ATOM_EMBED), lambda i: (0, 0)),
                pl.BlockSpec((ATOM_EMBED + N_RBF, HIDDEN), lambda i: (0, 0)),
                pl.BlockSpec((HIDDEN, 1), lambda i: (0, 0)),
                pl.BlockSpec((HIDDEN, HIDDEN), lambda i: (0, 0)),
                pl.BlockSpec((1, HIDDEN), lambda i: (0, 0)),
                pl.BlockSpec((1, 1), lambda i: (0, 0)),
                pl.BlockSpec((N_RBF, 1), lambda i: (0, 0)),
            ],
            out_specs=pl.BlockSpec((BE, HIDDEN), lambda i: (i, 0)),
            out_shape=jax.ShapeDtypeStruct((EH_PAD, HIDDEN), jnp.float32),
            compiler_params=pltpu.CompilerParams(
                dimension_semantics=("arbitrary",)),
        )(d2h, zch, at_pad, msg_W1, msg_b1.reshape(HIDDEN, 1), msg_W2,
          msg_b2.reshape(1, HIDDEN), gamma.reshape(1, 1), centers)

    msgs_a = edge_half(d2[:EH_PAD], zcol[:EH_PAD])
    msgs_b = edge_half(d2[E_HALF:E_HALF + EH_PAD],
                       zcol[E_HALF:E_HALF + EH_PAD])
    agg_a = _sc_scatter(row[:E_HALF], msgs_a)
    agg_b = _sc_scatter(row[E_HALF:], msgs_b)

    out = pl.pallas_call(
        _node_body,
        grid=(NB,),
        in_specs=[
            pl.BlockSpec((1, 1, BN), lambda i: (i, 0, 0)),
            pl.BlockSpec((NC, BN, HIDDEN), lambda i: (0, i, 0)),
            pl.BlockSpec((NC, BN, HIDDEN), lambda i: (0, i, 0)),
            pl.BlockSpec((1, 1, BN), lambda i: (i, 0, 0)),
            pl.BlockSpec((HIDDEN, ATOM_EMBED), lambda i: (0, 0)),
            pl.BlockSpec((ATOM_EMBED + HIDDEN, HIDDEN), lambda i: (0, 0)),
            pl.BlockSpec((HIDDEN, 1), lambda i: (0, 0)),
            pl.BlockSpec((HIDDEN, HIDDEN), lambda i: (0, 0)),
            pl.BlockSpec((1, HIDDEN), lambda i: (0, 0)),
            pl.BlockSpec((HIDDEN, HIDDEN // 2), lambda i: (0, 0)),
            pl.BlockSpec((1, HIDDEN // 2), lambda i: (0, 0)),
            pl.BlockSpec((HIDDEN // 2, 1), lambda i: (0, 0)),
            pl.BlockSpec((1, 1), lambda i: (0, 0)),
        ],
        out_specs=pl.BlockSpec((N_GRAPHS, 1), lambda i: (0, 0)),
        out_shape=jax.ShapeDtypeStruct((N_GRAPHS, 1), jnp.float32),
        scratch_shapes=[pltpu.VMEM((N_GRAPHS, HIDDEN), jnp.float32)],
        compiler_params=pltpu.CompilerParams(
            dimension_semantics=("arbitrary",)),
    )(z.reshape(NB, 1, BN), agg_a, agg_b, batch.reshape(NB, 1, BN), at_pad,
      node_W1, node_b1.reshape(HIDDEN, 1), node_W2,
      node_b2.reshape(1, HIDDEN), ro_W1, ro_b1.reshape(1, HIDDEN // 2),
      ro_W2, ro_b2.reshape(1, 1))

    return out.reshape(N_GRAPHS)


# speedup vs baseline: 1.0894x; 1.0291x over previous
import functools

import jax
import jax.numpy as jnp
from jax import lax
from jax.experimental import pallas as pl
from jax.experimental.pallas import tpu as pltpu
from jax.experimental.pallas import tpu_sc as plsc

N_NODES = 10000
N_EDGES = 320000
N_GRAPHS = 256
ATOM_EMBED = 64
HIDDEN = 128
N_RBF = 16
MAX_RADIUS = 5.0

E_PAD = 327680
NC = 2
NS = 16
NW = NC * NS
EPW = N_EDGES // NW
EPC = N_EDGES // NC
RPT = N_NODES // NS


def _sc_gather_body(row_h, col_h, z_h, px_h, py_h, pz_h, d2_h, zc_h,
                    row_v, col_v, z_v, px_v, py_v, pz_v, d2_v, zc_v):
    c = lax.axis_index("c")
    s = lax.axis_index("s")
    wid = s * NC + c
    base = wid * EPW
    pltpu.sync_copy(row_h.at[pl.ds(base, EPW)], row_v)
    pltpu.sync_copy(col_h.at[pl.ds(base, EPW)], col_v)
    pltpu.sync_copy(z_h, z_v)
    pltpu.sync_copy(px_h, px_v)
    pltpu.sync_copy(py_h, py_v)
    pltpu.sync_copy(pz_h, pz_v)

    def body(i, carry):
        sl = pl.ds(i * 16, 16)
        r = row_v[sl]
        cc = col_v[sl]
        ax = plsc.load_gather(px_v, [r])
        bx = plsc.load_gather(px_v, [cc])
        ay = plsc.load_gather(py_v, [r])
        by = plsc.load_gather(py_v, [cc])
        az = plsc.load_gather(pz_v, [r])
        bz = plsc.load_gather(pz_v, [cc])
        dx = ax - bx
        dy = ay - by
        dz = az - bz
        d2_v[sl] = dx * dx + dy * dy + dz * dz
        zc_v[sl] = plsc.load_gather(z_v, [cc])
        return carry

    lax.fori_loop(0, EPW // 16, body, 0)
    pltpu.sync_copy(d2_v, d2_h.at[pl.ds(base, EPW)])
    pltpu.sync_copy(zc_v, zc_h.at[pl.ds(base, EPW)])


@functools.cache
def _make_sc_gather():
    mesh = plsc.VectorSubcoreMesh(core_axis_name="c", subcore_axis_name="s")
    return functools.partial(
        pl.kernel,
        mesh=mesh,
        out_type=(jax.ShapeDtypeStruct((E_PAD,), jnp.float32),
                  jax.ShapeDtypeStruct((E_PAD,), jnp.int32)),
        scratch_types=[
            pltpu.VMEM((EPW,), jnp.int32),
            pltpu.VMEM((EPW,), jnp.int32),
            pltpu.VMEM((N_NODES,), jnp.int32),
            pltpu.VMEM((N_NODES,), jnp.float32),
            pltpu.VMEM((N_NODES,), jnp.float32),
            pltpu.VMEM((N_NODES,), jnp.float32),
            pltpu.VMEM((EPW,), jnp.float32),
            pltpu.VMEM((EPW,), jnp.int32),
        ],
        compiler_params=pltpu.CompilerParams(needs_layout_passes=False),
    )(_sc_gather_body)


def _sc_gather(row, col, z, px, py, pz):
    return _make_sc_gather()(row, col, z, px, py, pz)


E_HALF = N_EDGES // 2
EH_PAD = E_PAD // 2
SEPW = E_HALF // NW
_CHUNK = 128
_NFULL = SEPW // _CHUNK
_TAIL = SEPW - _NFULL * _CHUNK


def _sc_scatter_body(row_h, msg_h, agg_h, idx0, msg0, idx1, msg1, idx2, msg2,
                     idxt_v, msgt_v, si0, sm0, si1, sm1, si2, sm2, acc_sh):
    c = lax.axis_index("c")
    s = lax.axis_index("s")
    base = (c * NS + s) * SEPW
    bufs = ((idx0, msg0, si0, sm0), (idx1, msg1, si1, sm1),
            (idx2, msg2, si2, sm2))

    def zbody(r, carry):
        for q in range(HIDDEN // 16):
            msg0[r, pl.ds(q * 16, 16)] = jnp.zeros((16,), jnp.float32)
        return carry

    lax.fori_loop(0, _CHUNK, zbody, 0)
    off = 0
    for nrows in (128, 128, 128, 128, 112):
        pltpu.sync_copy(msg0.at[pl.ds(0, nrows)],
                        acc_sh.at[pl.ds(s * 624 + off, nrows)])
        off += nrows

    @pl.when(s == NS - 1)
    def _():
        pltpu.sync_copy(msg0.at[pl.ds(0, 16)],
                        acc_sh.at[pl.ds(N_NODES - 16, 16)])

    plsc.subcore_barrier()

    def _start(j, b):
        idx_b, msg_b, si, sm = b
        o = base + j * _CHUNK
        pltpu.async_copy(row_h.at[pl.ds(o, _CHUNK)], idx_b, si)
        pltpu.async_copy(msg_h.at[pl.ds(o, _CHUNK), :], msg_b, sm)

    def _wait(b):
        idx_b, msg_b, si, sm = b
        pltpu.make_async_copy(row_h.at[pl.ds(base, _CHUNK)], idx_b, si).wait()
        pltpu.make_async_copy(msg_h.at[pl.ds(base, _CHUNK), :], msg_b, sm).wait()

    _start(0, bufs[0])
    _start(1, bufs[1])
    _start(2, bufs[2])

    def body(jj, carry):
        for k in range(3):
            j = jj * 3 + k
            b = bufs[k]
            _wait(b)
            pltpu.sync_copy(b[1], acc_sh.at[b[0]], add=True)

            @pl.when(j + 3 < _NFULL)
            def _():
                _start(j + 3, b)
        return carry

    lax.fori_loop(0, _NFULL // 3, body, 0)
    o = base + _NFULL * _CHUNK
    pltpu.sync_copy(row_h.at[pl.ds(o, _TAIL)], idxt_v)
    pltpu.sync_copy(msg_h.at[pl.ds(o, _TAIL), :], msgt_v)
    pltpu.sync_copy(msgt_v, acc_sh.at[idxt_v], add=True)
    plsc.subcore_barrier()

    off = 0
    for nrows in (128, 128, 128, 128, 112):
        r0 = s * 624 + off
        pltpu.sync_copy(acc_sh.at[pl.ds(r0, nrows)], msg0.at[pl.ds(0, nrows)])
        pltpu.sync_copy(msg0.at[pl.ds(0, nrows)], agg_h.at[c, pl.ds(r0, nrows), :])
        off += nrows

    @pl.when(s == NS - 1)
    def _():
        r0 = N_NODES - 16
        pltpu.sync_copy(acc_sh.at[pl.ds(r0, 16)], msg0.at[pl.ds(0, 16)])
        pltpu.sync_copy(msg0.at[pl.ds(0, 16)], agg_h.at[c, pl.ds(r0, 16), :])


@functools.cache
def _make_sc_scatter():
    mesh = plsc.VectorSubcoreMesh(core_axis_name="c", subcore_axis_name="s")
    return functools.partial(
        pl.kernel,
        mesh=mesh,
        out_type=jax.ShapeDtypeStruct((NC, N_NODES, HIDDEN), jnp.float32),
        scratch_types=[
            pltpu.VMEM((_CHUNK,), jnp.int32),
            pltpu.VMEM((_CHUNK, HIDDEN), jnp.float32),
            pltpu.VMEM((_CHUNK,), jnp.int32),
            pltpu.VMEM((_CHUNK, HIDDEN), jnp.float32),
            pltpu.VMEM((_CHUNK,), jnp.int32),
            pltpu.VMEM((_CHUNK, HIDDEN), jnp.float32),
            pltpu.VMEM((_TAIL,), jnp.int32),
            pltpu.VMEM((_TAIL, HIDDEN), jnp.float32),
            pltpu.SemaphoreType.DMA,
            pltpu.SemaphoreType.DMA,
            pltpu.SemaphoreType.DMA,
            pltpu.SemaphoreType.DMA,
            pltpu.SemaphoreType.DMA,
            pltpu.SemaphoreType.DMA,
            pltpu.VMEM_SHARED((N_NODES, HIDDEN), jnp.float32),
        ],
        compiler_params=pltpu.CompilerParams(needs_layout_passes=False),
    )(_sc_scatter_body)


def _sc_scatter(row, msgs):
    return _make_sc_scatter()(row, msgs)


BE = 4096


def _edge_body(d2_ref, zc_ref, at_ref, mw1_ref, b1_ref, mw2_ref, b2_ref,
               gam_ref, cen_ref, msg_ref):
    d2 = d2_ref[...][None, :]
    elen = jnp.sqrt(d2 + 1e-12)
    diff = elen - cen_ref[...]
    rbfT = jnp.exp(-gam_ref[0, 0] * (diff * diff))
    zc = zc_ref[...][None, :]
    ohT = (zc == lax.broadcasted_iota(jnp.int32, (HIDDEN, BE), 0)
           ).astype(jnp.float32)
    xT = jnp.dot(at_ref[...].T, ohT, preferred_element_type=jnp.float32)
    msg_inT = jnp.concatenate([xT, rbfT], axis=0)
    pre = jnp.dot(mw1_ref[...].T, msg_inT,
                  preferred_element_type=jnp.float32)
    hT = jnp.maximum(pre + b1_ref[...], 0.0)
    out = lax.dot_general(hT, mw2_ref[...], (((0,), (0,)), ((), ())),
                          preferred_element_type=jnp.float32)
    msg_ref[...] = jnp.maximum(out + b2_ref[...], 0.0)


BN = 2000
NB = N_NODES // BN


def _node_body(z_ref, agg_ref, aggb_ref, bat_ref, at_ref, nw1_ref, b1_ref,
               nw2_ref, b2_ref, rw1_ref, rb1_ref, rw2_ref, rb2_ref,
               out_ref, mol_ref):
    i = pl.program_id(0)
    zr = z_ref[0]
    ohzT = (zr == lax.broadcasted_iota(jnp.int32, (HIDDEN, BN), 0)
            ).astype(jnp.float32)
    xT = jnp.dot(at_ref[...].T, ohzT, preferred_element_type=jnp.float32)
    aggT = ((agg_ref[0] + agg_ref[1])
            + (aggb_ref[0] + aggb_ref[1])).T
    nfT = jnp.concatenate([xT, aggT], axis=0)
    pre = jnp.dot(nw1_ref[...].T, nfT, preferred_element_type=jnp.float32)
    h2T = jnp.maximum(pre + b1_ref[...], 0.0)
    nout = lax.dot_general(h2T, nw2_ref[...], (((0,), (0,)), ((), ())),
                           preferred_element_type=jnp.float32)
    nout = jnp.maximum(nout + b2_ref[...], 0.0)
    bt = bat_ref[0]
    ohb = (lax.broadcasted_iota(jnp.int32, (N_GRAPHS, BN), 0) == bt
           ).astype(jnp.float32)
    part = jnp.dot(ohb, nout, preferred_element_type=jnp.float32, precision=lax.Precision.HIGHEST)

    @pl.when(i == 0)
    def _():
        mol_ref[...] = part

    @pl.when(i > 0)
    def _():
        mol_ref[...] = mol_ref[...] + part

    @pl.when(i == NB - 1)
    def _():
        h3 = jnp.dot(mol_ref[...], rw1_ref[...],
                     preferred_element_type=jnp.float32)
        h3 = jnp.maximum(h3 + rb1_ref[...], 0.0)
        o = jnp.dot(h3, rw2_ref[...], preferred_element_type=jnp.float32)
        out_ref[...] = o + rb2_ref[...]


def kernel(pos, z, batch, edge_index, atom_table, gamma,
           msg_W1, msg_b1, msg_W2, msg_b2,
           node_W1, node_b1, node_W2, node_b2,
           ro_W1, ro_b1, ro_W2, ro_b2):
    row = edge_index[0]
    col = edge_index[1]
    px = pos[:, 0]
    py = pos[:, 1]
    pz = pos[:, 2]

    d2, zcol = _sc_gather(row, col, z, px, py, pz)

    at_pad = jnp.pad(atom_table, ((0, HIDDEN - atom_table.shape[0]), (0, 0)))
    centers = jnp.linspace(0.0, MAX_RADIUS, N_RBF,
                           dtype=jnp.float32).reshape(N_RBF, 1)

    def edge_half(d2h, zch):
        return pl.pallas_call(
            _edge_body,
            grid=(EH_PAD // BE,),
            in_specs=[
                pl.BlockSpec((BE,), lambda i: (i,)),
                pl.BlockSpec((BE,), lambda i: (i,)),
                pl.BlockSpec((HIDDEN, ATOM_EMBED), lambda i: (0, 0)),
                pl.BlockSpec((ATOM_EMBED + N_RBF, HIDDEN), lambda i: (0, 0)),
                pl.BlockSpec((HIDDEN, 1), lambda i: (0, 0)),
                pl.BlockSpec((HIDDEN, HIDDEN), lambda i: (0, 0)),
                pl.BlockSpec((1, HIDDEN), lambda i: (0, 0)),
                pl.BlockSpec((1, 1), lambda i: (0, 0)),
                pl.BlockSpec((N_RBF, 1), lambda i: (0, 0)),
            ],
            out_specs=pl.BlockSpec((BE, HIDDEN), lambda i: (i, 0)),
            out_shape=jax.ShapeDtypeStruct((EH_PAD, HIDDEN), jnp.float32),
            compiler_params=pltpu.CompilerParams(
                dimension_semantics=("arbitrary",)),
        )(d2h, zch, at_pad, msg_W1, msg_b1.reshape(HIDDEN, 1), msg_W2,
          msg_b2.reshape(1, HIDDEN), gamma.reshape(1, 1), centers)

    msgs_a = edge_half(d2[:EH_PAD], zcol[:EH_PAD])
    msgs_b = edge_half(d2[E_HALF:E_HALF + EH_PAD],
                       zcol[E_HALF:E_HALF + EH_PAD])
    agg_a = _sc_scatter(row[:E_HALF], msgs_a)
    agg_b = _sc_scatter(row[E_HALF:], msgs_b)

    out = pl.pallas_call(
        _node_body,
        grid=(NB,),
        in_specs=[
            pl.BlockSpec((1, 1, BN), lambda i: (i, 0, 0)),
            pl.BlockSpec((NC, BN, HIDDEN), lambda i: (0, i, 0)),
            pl.BlockSpec((NC, BN, HIDDEN), lambda i: (0, i, 0)),
            pl.BlockSpec((1, 1, BN), lambda i: (i, 0, 0)),
            pl.BlockSpec((HIDDEN, ATOM_EMBED), lambda i: (0, 0)),
            pl.BlockSpec((ATOM_EMBED + HIDDEN, HIDDEN), lambda i: (0, 0)),
            pl.BlockSpec((HIDDEN, 1), lambda i: (0, 0)),
            pl.BlockSpec((HIDDEN, HIDDEN), lambda i: (0, 0)),
            pl.BlockSpec((1, HIDDEN), lambda i: (0, 0)),
            pl.BlockSpec((HIDDEN, HIDDEN // 2), lambda i: (0, 0)),
            pl.BlockSpec((1, HIDDEN // 2), lambda i: (0, 0)),
            pl.BlockSpec((HIDDEN // 2, 1), lambda i: (0, 0)),
            pl.BlockSpec((1, 1), lambda i: (0, 0)),
        ],
        out_specs=pl.BlockSpec((N_GRAPHS, 1), lambda i: (0, 0)),
        out_shape=jax.ShapeDtypeStruct((N_GRAPHS, 1), jnp.float32),
        scratch_shapes=[pltpu.VMEM((N_GRAPHS, HIDDEN), jnp.float32)],
        compiler_params=pltpu.CompilerParams(
            dimension_semantics=("arbitrary",)),
    )(z.reshape(NB, 1, BN), agg_a, agg_b, batch.reshape(NB, 1, BN), at_pad,
      node_W1, node_b1.reshape(HIDDEN, 1), node_W2,
      node_b2.reshape(1, HIDDEN), ro_W1, ro_b1.reshape(1, HIDDEN // 2),
      ro_W2, ro_b2.reshape(1, 1))

    return out.reshape(N_GRAPHS)


# speedup vs baseline: 1.0896x; 1.0001x over previous
import functools

import jax
import jax.numpy as jnp
from jax import lax
from jax.experimental import pallas as pl
from jax.experimental.pallas import tpu as pltpu
from jax.experimental.pallas import tpu_sc as plsc

N_NODES = 10000
N_EDGES = 320000
N_GRAPHS = 256
ATOM_EMBED = 64
HIDDEN = 128
N_RBF = 16
MAX_RADIUS = 5.0

E_PAD = 327680
NC = 2
NS = 16
NW = NC * NS
EPW = N_EDGES // NW


def _sc_gather_body(row_h, col_h, z_h, px_h, py_h, pz_h, d2_h, zc_h,
                    row_v, col_v, z_v, px_v, py_v, pz_v, d2_v, zc_v):
    c = lax.axis_index("c")
    s = lax.axis_index("s")
    wid = s * NC + c
    base = wid * EPW
    pltpu.sync_copy(row_h.at[pl.ds(base, EPW)], row_v)
    pltpu.sync_copy(col_h.at[pl.ds(base, EPW)], col_v)
    pltpu.sync_copy(z_h, z_v)
    pltpu.sync_copy(px_h, px_v)
    pltpu.sync_copy(py_h, py_v)
    pltpu.sync_copy(pz_h, pz_v)

    def body(i, carry):
        sl = pl.ds(i * 16, 16)
        r = row_v[sl]
        cc = col_v[sl]
        ax = plsc.load_gather(px_v, [r])
        bx = plsc.load_gather(px_v, [cc])
        ay = plsc.load_gather(py_v, [r])
        by = plsc.load_gather(py_v, [cc])
        az = plsc.load_gather(pz_v, [r])
        bz = plsc.load_gather(pz_v, [cc])
        dx = ax - bx
        dy = ay - by
        dz = az - bz
        d2_v[sl] = dx * dx + dy * dy + dz * dz
        zc_v[sl] = plsc.load_gather(z_v, [cc])
        return carry

    lax.fori_loop(0, EPW // 16, body, 0)
    pltpu.sync_copy(d2_v, d2_h.at[pl.ds(base, EPW)])
    pltpu.sync_copy(zc_v, zc_h.at[pl.ds(base, EPW)])


@functools.cache
def _make_sc_gather():
    mesh = plsc.VectorSubcoreMesh(core_axis_name="c", subcore_axis_name="s")
    return functools.partial(
        pl.kernel,
        mesh=mesh,
        out_type=(jax.ShapeDtypeStruct((E_PAD,), jnp.float32),
                  jax.ShapeDtypeStruct((E_PAD,), jnp.int32)),
        scratch_types=[
            pltpu.VMEM((EPW,), jnp.int32),
            pltpu.VMEM((EPW,), jnp.int32),
            pltpu.VMEM((N_NODES,), jnp.int32),
            pltpu.VMEM((N_NODES,), jnp.float32),
            pltpu.VMEM((N_NODES,), jnp.float32),
            pltpu.VMEM((N_NODES,), jnp.float32),
            pltpu.VMEM((EPW,), jnp.float32),
            pltpu.VMEM((EPW,), jnp.int32),
        ],
        compiler_params=pltpu.CompilerParams(needs_layout_passes=False),
    )(_sc_gather_body)


def _sc_gather(row, col, z, px, py, pz):
    return _make_sc_gather()(row, col, z, px, py, pz)


E_HALF = N_EDGES // 2
EH_PAD = E_PAD // 2
SEPW = E_HALF // NW
_CHUNK = 128
_NFULL = SEPW // _CHUNK
_TAIL = SEPW - _NFULL * _CHUNK


def _sc_scatter_body(row_h, msg_h, agg_h, idx0, msg0, idx1, msg1, idx2, msg2,
                     idxt_v, msgt_v, si0, sm0, si1, sm1, si2, sm2, acc_sh):
    c = lax.axis_index("c")
    s = lax.axis_index("s")
    base = (c * NS + s) * SEPW
    bufs = ((idx0, msg0, si0, sm0), (idx1, msg1, si1, sm1),
            (idx2, msg2, si2, sm2))

    def zbody(r, carry):
        for q in range(HIDDEN // 16):
            msg0[r, pl.ds(q * 16, 16)] = jnp.zeros((16,), jnp.float32)
        return carry

    lax.fori_loop(0, _CHUNK, zbody, 0)
    off = 0
    for nrows in (128, 128, 128, 128, 112):
        pltpu.sync_copy(msg0.at[pl.ds(0, nrows)],
                        acc_sh.at[pl.ds(s * 624 + off, nrows)])
        off += nrows

    @pl.when(s == NS - 1)
    def _():
        pltpu.sync_copy(msg0.at[pl.ds(0, 16)],
                        acc_sh.at[pl.ds(N_NODES - 16, 16)])

    plsc.subcore_barrier()

    def _start(j, b):
        idx_b, msg_b, si, sm = b
        o = base + j * _CHUNK
        pltpu.async_copy(row_h.at[pl.ds(o, _CHUNK)], idx_b, si)
        pltpu.async_copy(msg_h.at[pl.ds(o, _CHUNK), :], msg_b, sm)

    def _wait(b):
        idx_b, msg_b, si, sm = b
        pltpu.make_async_copy(row_h.at[pl.ds(base, _CHUNK)], idx_b, si).wait()
        pltpu.make_async_copy(msg_h.at[pl.ds(base, _CHUNK), :], msg_b, sm).wait()

    _start(0, bufs[0])
    _start(1, bufs[1])
    _start(2, bufs[2])

    def body(jj, carry):
        for k in range(3):
            j = jj * 3 + k
            b = bufs[k]
            _wait(b)
            pltpu.sync_copy(b[1], acc_sh.at[b[0]], add=True)

            @pl.when(j + 3 < _NFULL)
            def _():
                _start(j + 3, b)
        return carry

    lax.fori_loop(0, _NFULL // 3, body, 0)
    o = base + _NFULL * _CHUNK
    pltpu.sync_copy(row_h.at[pl.ds(o, _TAIL)], idxt_v)
    pltpu.sync_copy(msg_h.at[pl.ds(o, _TAIL), :], msgt_v)
    pltpu.sync_copy(msgt_v, acc_sh.at[idxt_v], add=True)
    plsc.subcore_barrier()

    off = 0
    for nrows in (128, 128, 128, 128, 112):
        r0 = s * 624 + off
        pltpu.sync_copy(acc_sh.at[pl.ds(r0, nrows)], msg0.at[pl.ds(0, nrows)])
        pltpu.sync_copy(msg0.at[pl.ds(0, nrows)], agg_h.at[c, pl.ds(r0, nrows), :])
        off += nrows

    @pl.when(s == NS - 1)
    def _():
        r0 = N_NODES - 16
        pltpu.sync_copy(acc_sh.at[pl.ds(r0, 16)], msg0.at[pl.ds(0, 16)])
        pltpu.sync_copy(msg0.at[pl.ds(0, 16)], agg_h.at[c, pl.ds(r0, 16), :])


@functools.cache
def _make_sc_scatter():
    mesh = plsc.VectorSubcoreMesh(core_axis_name="c", subcore_axis_name="s")
    return functools.partial(
        pl.kernel,
        mesh=mesh,
        out_type=jax.ShapeDtypeStruct((NC, N_NODES, HIDDEN), jnp.float32),
        scratch_types=[
            pltpu.VMEM((_CHUNK,), jnp.int32),
            pltpu.VMEM((_CHUNK, HIDDEN), jnp.float32),
            pltpu.VMEM((_CHUNK,), jnp.int32),
            pltpu.VMEM((_CHUNK, HIDDEN), jnp.float32),
            pltpu.VMEM((_CHUNK,), jnp.int32),
            pltpu.VMEM((_CHUNK, HIDDEN), jnp.float32),
            pltpu.VMEM((_TAIL,), jnp.int32),
            pltpu.VMEM((_TAIL, HIDDEN), jnp.float32),
            pltpu.SemaphoreType.DMA,
            pltpu.SemaphoreType.DMA,
            pltpu.SemaphoreType.DMA,
            pltpu.SemaphoreType.DMA,
            pltpu.SemaphoreType.DMA,
            pltpu.SemaphoreType.DMA,
            pltpu.VMEM_SHARED((N_NODES, HIDDEN), jnp.float32),
        ],
        compiler_params=pltpu.CompilerParams(needs_layout_passes=False),
    )(_sc_scatter_body)


def _sc_scatter(row, msgs):
    return _make_sc_scatter()(row, msgs)


BE = 4096


def _edge_body(d2_ref, zc_ref, at_ref, mw1_ref, b1_ref, mw2_ref, b2_ref,
               gam_ref, cen_ref, msg_ref):
    d2 = d2_ref[...][None, :]
    elen = jnp.sqrt(d2 + 1e-12)
    diff = elen - cen_ref[...]
    rbfT = jnp.exp(-gam_ref[0, 0] * (diff * diff))
    zc = zc_ref[...][None, :]
    ohT = (zc == lax.broadcasted_iota(jnp.int32, (HIDDEN, BE), 0)
           ).astype(jnp.float32)
    xT = jnp.dot(at_ref[...].T, ohT, preferred_element_type=jnp.float32)
    msg_inT = jnp.concatenate([xT, rbfT], axis=0)
    pre = jnp.dot(mw1_ref[...].T, msg_inT,
                  preferred_element_type=jnp.float32)
    hT = jnp.maximum(pre + b1_ref[...], 0.0)
    out = lax.dot_general(hT, mw2_ref[...], (((0,), (0,)), ((), ())),
                          preferred_element_type=jnp.float32)
    msg_ref[...] = jnp.maximum(out + b2_ref[...], 0.0)


BN = 2000
NB = N_NODES // BN


def _node_body(z_ref, agg_ref, aggb_ref, bat_ref, at_ref, nw1_ref, b1_ref,
               nw2_ref, b2_ref, rw1_ref, rb1_ref, rw2_ref, rb2_ref,
               out_ref, mol_ref):
    i = pl.program_id(0)
    zr = z_ref[0]
    ohzT = (zr == lax.broadcasted_iota(jnp.int32, (HIDDEN, BN), 0)
            ).astype(jnp.float32)
    xT = jnp.dot(at_ref[...].T, ohzT, preferred_element_type=jnp.float32)
    aggT = ((agg_ref[0] + agg_ref[1])
            + (aggb_ref[0] + aggb_ref[1])).T
    nfT = jnp.concatenate([xT, aggT], axis=0)
    pre = jnp.dot(nw1_ref[...].T, nfT, preferred_element_type=jnp.float32)
    h2T = jnp.maximum(pre + b1_ref[...], 0.0)
    nout = lax.dot_general(h2T, nw2_ref[...], (((0,), (0,)), ((), ())),
                           preferred_element_type=jnp.float32)
    nout = jnp.maximum(nout + b2_ref[...], 0.0)
    bt = bat_ref[0]
    ohb = (lax.broadcasted_iota(jnp.int32, (N_GRAPHS, BN), 0) == bt
           ).astype(jnp.float32)
    part = jnp.dot(ohb, nout, preferred_element_type=jnp.float32, precision=lax.Precision.HIGHEST)

    @pl.when(i == 0)
    def _():
        mol_ref[...] = part

    @pl.when(i > 0)
    def _():
        mol_ref[...] = mol_ref[...] + part

    @pl.when(i == NB - 1)
    def _():
        h3 = jnp.dot(mol_ref[...], rw1_ref[...],
                     preferred_element_type=jnp.float32)
        h3 = jnp.maximum(h3 + rb1_ref[...], 0.0)
        o = jnp.dot(h3, rw2_ref[...], preferred_element_type=jnp.float32)
        out_ref[...] = o + rb2_ref[...]


def kernel(pos, z, batch, edge_index, atom_table, gamma,
           msg_W1, msg_b1, msg_W2, msg_b2,
           node_W1, node_b1, node_W2, node_b2,
           ro_W1, ro_b1, ro_W2, ro_b2):
    row = edge_index[0]
    col = edge_index[1]
    px = pos[:, 0]
    py = pos[:, 1]
    pz = pos[:, 2]

    d2, zcol = _sc_gather(row, col, z, px, py, pz)

    at_pad = jnp.pad(atom_table, ((0, HIDDEN - atom_table.shape[0]), (0, 0)))
    centers = jnp.linspace(0.0, MAX_RADIUS, N_RBF,
                           dtype=jnp.float32).reshape(N_RBF, 1)

    def edge_half(d2h, zch):
        return pl.pallas_call(
            _edge_body,
            grid=(EH_PAD // BE,),
            in_specs=[
                pl.BlockSpec((BE,), lambda i: (i,)),
                pl.BlockSpec((BE,), lambda i: (i,)),
                pl.BlockSpec((HIDDEN, ATOM_EMBED), lambda i: (0, 0)),
                pl.BlockSpec((ATOM_EMBED + N_RBF, HIDDEN), lambda i: (0, 0)),
                pl.BlockSpec((HIDDEN, 1), lambda i: (0, 0)),
                pl.BlockSpec((HIDDEN, HIDDEN), lambda i: (0, 0)),
                pl.BlockSpec((1, HIDDEN), lambda i: (0, 0)),
                pl.BlockSpec((1, 1), lambda i: (0, 0)),
                pl.BlockSpec((N_RBF, 1), lambda i: (0, 0)),
            ],
            out_specs=pl.BlockSpec((BE, HIDDEN), lambda i: (i, 0)),
            out_shape=jax.ShapeDtypeStruct((EH_PAD, HIDDEN), jnp.float32),
            compiler_params=pltpu.CompilerParams(
                dimension_semantics=("arbitrary",)),
        )(d2h, zch, at_pad, msg_W1, msg_b1.reshape(HIDDEN, 1), msg_W2,
          msg_b2.reshape(1, HIDDEN), gamma.reshape(1, 1), centers)

    msgs_a = edge_half(d2[:EH_PAD], zcol[:EH_PAD])
    msgs_b = edge_half(d2[E_HALF:E_HALF + EH_PAD],
                       zcol[E_HALF:E_HALF + EH_PAD])
    agg_a = _sc_scatter(row[:E_HALF], msgs_a)
    agg_b = _sc_scatter(row[E_HALF:], msgs_b)

    out = pl.pallas_call(
        _node_body,
        grid=(NB,),
        in_specs=[
            pl.BlockSpec((1, 1, BN), lambda i: (i, 0, 0)),
            pl.BlockSpec((NC, BN, HIDDEN), lambda i: (0, i, 0)),
            pl.BlockSpec((NC, BN, HIDDEN), lambda i: (0, i, 0)),
            pl.BlockSpec((1, 1, BN), lambda i: (i, 0, 0)),
            pl.BlockSpec((HIDDEN, ATOM_EMBED), lambda i: (0, 0)),
            pl.BlockSpec((ATOM_EMBED + HIDDEN, HIDDEN), lambda i: (0, 0)),
            pl.BlockSpec((HIDDEN, 1), lambda i: (0, 0)),
            pl.BlockSpec((HIDDEN, HIDDEN), lambda i: (0, 0)),
            pl.BlockSpec((1, HIDDEN), lambda i: (0, 0)),
            pl.BlockSpec((HIDDEN, HIDDEN // 2), lambda i: (0, 0)),
            pl.BlockSpec((1, HIDDEN // 2), lambda i: (0, 0)),
            pl.BlockSpec((HIDDEN // 2, 1), lambda i: (0, 0)),
            pl.BlockSpec((1, 1), lambda i: (0, 0)),
        ],
        out_specs=pl.BlockSpec((N_GRAPHS, 1), lambda i: (0, 0)),
        out_shape=jax.ShapeDtypeStruct((N_GRAPHS, 1), jnp.float32),
        scratch_shapes=[pltpu.VMEM((N_GRAPHS, HIDDEN), jnp.float32)],
        compiler_params=pltpu.CompilerParams(
            dimension_semantics=("arbitrary",)),
    )(z.reshape(NB, 1, BN), agg_a, agg_b, batch.reshape(NB, 1, BN), at_pad,
      node_W1, node_b1.reshape(HIDDEN, 1), node_W2,
      node_b2.reshape(1, HIDDEN), ro_W1, ro_b1.reshape(1, HIDDEN // 2),
      ro_W2, ro_b2.reshape(1, 1))

    return out.reshape(N_GRAPHS)
